# baseline (speedup 1.0000x reference)
import functools

import jax
import jax.numpy as jnp
from jax import lax
from jax.experimental import pallas as pl
from jax.experimental.pallas import tpu as pltpu
from jax.experimental.pallas import tpu_sc as plsc

_VOCAB = 1000000
_EMBED = 64
_MAXLEN = 64
_SEGN = 2
_BATCH = 4096
_SEQ = 64

_HALF = 1 << 19
_TPB = 4096

_NC = 2
_NS = 16
_NW = _NC * _NS
_ROWS = _BATCH * _SEQ
_RPW = _ROWS // _NW
_C = 256
_NCHUNK = _RPW // _C
_G = 128
_NG = _C // _G


def _tp_body(a_ref, b_ref, o_ref):
  for k in range(_TPB // 128):
    d = pl.ds(k * 128, 128)
    o_ref[d, 0:_EMBED] = jnp.transpose(a_ref[:, d])
    o_ref[d, _EMBED:2 * _EMBED] = jnp.transpose(b_ref[:, d])


def _pack_table(wt):
  nblk = _HALF // _TPB
  return pl.pallas_call(
      _tp_body,
      grid=(nblk,),
      in_specs=[
          pl.BlockSpec((_EMBED, _TPB), lambda i: (0, i)),
          pl.BlockSpec((_EMBED, _TPB), lambda i: (0, i + nblk)),
      ],
      out_specs=pl.BlockSpec((_TPB, 2 * _EMBED), lambda i: (i, 0)),
      out_shape=jax.ShapeDtypeStruct((_HALF, 2 * _EMBED), jnp.float32),
  )(wt, wt)


def _sc_body(ids_hbm, seg_hbm, word_hbm, pos_hbm, segtab_hbm, out_hbm,
             idsw_v, segw_v, idxg0_v, idxg1_v, gbuf0_v, gbuf1_v, obuf_v,
             pos_v, segtab_v, sem0, sem1):
  idxg = (idxg0_v, idxg1_v)
  gbuf = (gbuf0_v, gbuf1_v)
  wid = lax.axis_index("c") * _NS + lax.axis_index("s")
  base = wid * _RPW

  t0 = pl.multiple_of(wid * _RPW, _RPW)
  pltpu.sync_copy(ids_hbm.at[:, pl.ds(t0, _RPW)], idsw_v)
  pltpu.sync_copy(seg_hbm.at[:, pl.ds(t0, _RPW)], segw_v)
  pltpu.sync_copy(pos_hbm.at[:, pl.ds(pl.multiple_of(wid * 2 * _EMBED, 128),
                                      2 * _EMBED)], pos_v)
  pltpu.sync_copy(segtab_hbm, segtab_v)

  seg0 = [segtab_v[0, pl.ds(o * 16, 16)] for o in range(4)]
  dseg = [segtab_v[0, pl.ds(_EMBED + o * 16, 16)] - seg0[o] for o in range(4)]

  def fill_idx(c, slot):
    for jr in range(_NG):
      def body(t, _, jr=jr):
        d = pl.ds(c * _C + jr * _G + t * 16, 16)
        idxg[slot][jr, pl.ds(t * 16, 16)] = lax.bitwise_and(
            idsw_v[0, d], _HALF - 1)
        return _
      lax.fori_loop(0, _G // 16, body, None)

  def fire(c, slot, sem):
    fill_idx(c, slot)
    return [pltpu.async_copy(word_hbm.at[idxg[slot].at[j]],
                             gbuf[slot].at[pl.ds(j * _G, _G)], sem)
            for j in range(_NG)]

  def compute(c, slot):
    hp = lax.mul(lax.div(c, _NCHUNK // 2), _EMBED)
    basev = [pos_v[0, pl.ds(hp + o * 16, 16)] + seg0[o] for o in range(4)]

    def grp(t, _):
      d = pl.ds(c * _C + t * 16, 16)
      ids16 = idsw_v[0, d]
      seg16 = segw_v[0, d]
      for i in range(16):
        r = t * 16 + i
        h64 = lax.bitwise_and(lax.shift_right_logical(ids16[i], 13), _EMBED)
        gf = lax.convert_element_type(seg16[i], jnp.float32)
        gfv = lax.broadcast(gf, (16,))
        q = t * 8 + (i // 2)
        ocol = (i % 2) * _EMBED
        for jj in range(4):
          o = jj * 16
          v = gbuf[slot][r, pl.ds(h64 + o, 16)]
          obuf_v[q, pl.ds(ocol + o, 16)] = v + basev[jj] + gfv * dseg[jj]
      return _
    lax.fori_loop(0, _C // 16, grp, None)

    ob = pl.multiple_of((base + c * _C) // 2, _C // 2)
    pltpu.sync_copy(obuf_v, out_hbm.at[pl.ds(ob, _C // 2)])

  def pair(p, _):
    c0 = p * 2
    cps0 = fire(c0, 0, sem0)
    cps1 = fire(c0 + 1, 1, sem1)
    for cp in cps0:
      cp.wait()
    compute(c0, 0)
    for cp in cps1:
      cp.wait()
    compute(c0 + 1, 1)
    return _

  lax.fori_loop(0, _NCHUNK // 2, pair, None)


@functools.partial(
    pl.kernel,
    out_type=jax.ShapeDtypeStruct((_ROWS // 2, 2 * _EMBED), jnp.float32),
    mesh=plsc.VectorSubcoreMesh(core_axis_name="c", subcore_axis_name="s"),
    scratch_types=[
        pltpu.VMEM((1, _RPW), jnp.int32),
        pltpu.VMEM((1, _RPW), jnp.int32),
        pltpu.VMEM((_NG, _G), jnp.int32),
        pltpu.VMEM((_NG, _G), jnp.int32),
        pltpu.VMEM((_C, 2 * _EMBED), jnp.float32),
        pltpu.VMEM((_C, 2 * _EMBED), jnp.float32),
        pltpu.VMEM((_C // 2, 2 * _EMBED), jnp.float32),
        pltpu.VMEM((1, 2 * _EMBED), jnp.float32),
        pltpu.VMEM((1, 2 * _EMBED), jnp.float32),
        pltpu.SemaphoreType.DMA,
        pltpu.SemaphoreType.DMA,
    ],
    compiler_params=pltpu.CompilerParams(use_tc_tiling_on_sc=True),
)
def _embed_sc(*refs):
  _sc_body(*refs)


@jax.jit
def kernel(input_ids, seg_ids, word_embed, pos_embed, seg_embed):
  ids1 = input_ids.astype(jnp.int32).T.reshape(1, _ROWS)
  seg1 = seg_ids.astype(jnp.int32).T.reshape(1, _ROWS)
  wt = jnp.pad(word_embed.T, ((0, 0), (0, 2 * _HALF - _VOCAB)))
  word2 = jnp.concatenate([wt[:, :_HALF].T, wt[:, _HALF:].T], axis=1)
  pos3 = pos_embed.reshape(1, _MAXLEN * _EMBED)
  segtab2 = seg_embed.reshape(1, 2 * _EMBED)
  out2 = _embed_sc(ids1, seg1, word2, pos3, segtab2)
  return jnp.transpose(out2.reshape(_SEQ, _BATCH, _EMBED), (1, 0, 2))


# speedup vs baseline: 1.5523x; 1.5523x over previous
import functools

import jax
import jax.numpy as jnp
from jax import lax
from jax.experimental import pallas as pl
from jax.experimental.pallas import tpu as pltpu
from jax.experimental.pallas import tpu_sc as plsc

_VOCAB = 1000000
_EMBED = 64
_MAXLEN = 64
_SEGN = 2
_BATCH = 4096
_SEQ = 64

_WHALF = 512000
_BSTART = 488000
_TPB = 4096

_NC = 2
_NS = 16
_NW = _NC * _NS
_ROWS = _BATCH * _SEQ
_RPW = _ROWS // _NW
_C = 256
_NCHUNK = _RPW // _C
_G = 128
_NG = _C // _G


def _tp_body(a_ref, b_ref, e_ref, o_ref):
  dn = (((0,), (0,)), ((), ()))
  o_ref[:, 0:_EMBED] = lax.dot_general(
      a_ref[...], e_ref[...], dn, preferred_element_type=jnp.float32)
  o_ref[:, _EMBED:2 * _EMBED] = lax.dot_general(
      b_ref[...], e_ref[...], dn, preferred_element_type=jnp.float32)


def _pack_table(wt):
  nblk = _WHALF // _TPB
  wtb = lax.slice(wt, (0, _BSTART), (_EMBED, _VOCAB))
  return pl.pallas_call(
      _tp_body,
      grid=(nblk,),
      in_specs=[
          pl.BlockSpec((_EMBED, _TPB), lambda i: (0, i)),
          pl.BlockSpec((_EMBED, _TPB), lambda i: (0, i)),
          pl.BlockSpec((_EMBED, _EMBED), lambda i: (0, 0)),
      ],
      out_specs=pl.BlockSpec((_TPB, 2 * _EMBED), lambda i: (i, 0)),
      out_shape=jax.ShapeDtypeStruct((_WHALF, 2 * _EMBED), jnp.float32),
  )(wt, wtb, jnp.eye(_EMBED, dtype=jnp.float32))


def _sc_body(ids_hbm, seg_hbm, word_hbm, pos_hbm, segtab_hbm, out_hbm,
             idsw_v, segw_v, idxg0_v, idxg1_v, gbuf0_v, gbuf1_v, obuf_v,
             pos_v, segtab_v, sem0, sem1):
  idxg = (idxg0_v, idxg1_v)
  gbuf = (gbuf0_v, gbuf1_v)
  wid = lax.axis_index("c") * _NS + lax.axis_index("s")
  base = wid * _RPW

  t0 = pl.multiple_of(wid * _RPW, _RPW)
  pltpu.sync_copy(ids_hbm.at[:, pl.ds(t0, _RPW)], idsw_v)
  pltpu.sync_copy(seg_hbm.at[:, pl.ds(t0, _RPW)], segw_v)
  pltpu.sync_copy(pos_hbm.at[:, pl.ds(pl.multiple_of(wid * 2 * _EMBED, 128),
                                      2 * _EMBED)], pos_v)
  pltpu.sync_copy(segtab_hbm, segtab_v)

  seg0 = [segtab_v[0, pl.ds(o * 16, 16)] for o in range(4)]
  dseg = [segtab_v[0, pl.ds(_EMBED + o * 16, 16)] - seg0[o] for o in range(4)]

  def fill_idx(c, slot):
    for jr in range(_NG):
      def body(t, _, jr=jr):
        d = pl.ds(c * _C + jr * _G + t * 16, 16)
        v = idsw_v[0, d]
        idxg[slot][jr, pl.ds(t * 16, 16)] = jnp.where(
            v >= _WHALF, v - _BSTART, v)
        return _
      lax.fori_loop(0, _G // 16, body, None)

  def fire(c, slot, sem):
    fill_idx(c, slot)
    return [pltpu.async_copy(word_hbm.at[idxg[slot].at[j]],
                             gbuf[slot].at[pl.ds(j * _G, _G)], sem)
            for j in range(_NG)]

  def compute(c, slot):
    hp = lax.mul(lax.div(c, _NCHUNK // 2), _EMBED)
    basev = [pos_v[0, pl.ds(hp + o * 16, 16)] + seg0[o] for o in range(4)]

    def grp(t, _):
      d = pl.ds(c * _C + t * 16, 16)
      ids16 = idsw_v[0, d]
      seg16 = segw_v[0, d]
      for i in range(16):
        r = t * 16 + i
        h64 = lax.mul(lax.convert_element_type(ids16[i] >= _WHALF,
                                               jnp.int32), _EMBED)
        gf = lax.convert_element_type(seg16[i], jnp.float32)
        gfv = lax.broadcast(gf, (16,))
        q = t * 8 + (i // 2)
        ocol = (i % 2) * _EMBED
        for jj in range(4):
          o = jj * 16
          v = gbuf[slot][r, pl.ds(h64 + o, 16)]
          obuf_v[q, pl.ds(ocol + o, 16)] = v + basev[jj] + gfv * dseg[jj]
      return _
    lax.fori_loop(0, _C // 16, grp, None)

    ob = pl.multiple_of((base + c * _C) // 2, _C // 2)
    pltpu.sync_copy(obuf_v, out_hbm.at[pl.ds(ob, _C // 2)])

  def pair(p, _):
    c0 = p * 2
    cps0 = fire(c0, 0, sem0)
    cps1 = fire(c0 + 1, 1, sem1)
    for cp in cps0:
      cp.wait()
    compute(c0, 0)
    for cp in cps1:
      cp.wait()
    compute(c0 + 1, 1)
    return _

  lax.fori_loop(0, _NCHUNK // 2, pair, None)


@functools.partial(
    pl.kernel,
    out_type=jax.ShapeDtypeStruct((_ROWS // 2, 2 * _EMBED), jnp.float32),
    mesh=plsc.VectorSubcoreMesh(core_axis_name="c", subcore_axis_name="s"),
    scratch_types=[
        pltpu.VMEM((1, _RPW), jnp.int32),
        pltpu.VMEM((1, _RPW), jnp.int32),
        pltpu.VMEM((_NG, _G), jnp.int32),
        pltpu.VMEM((_NG, _G), jnp.int32),
        pltpu.VMEM((_C, 2 * _EMBED), jnp.float32),
        pltpu.VMEM((_C, 2 * _EMBED), jnp.float32),
        pltpu.VMEM((_C // 2, 2 * _EMBED), jnp.float32),
        pltpu.VMEM((1, 2 * _EMBED), jnp.float32),
        pltpu.VMEM((1, 2 * _EMBED), jnp.float32),
        pltpu.SemaphoreType.DMA,
        pltpu.SemaphoreType.DMA,
    ],
    compiler_params=pltpu.CompilerParams(use_tc_tiling_on_sc=True),
)
def _embed_sc(*refs):
  _sc_body(*refs)


@jax.jit
def kernel(input_ids, seg_ids, word_embed, pos_embed, seg_embed):
  ids1 = input_ids.astype(jnp.int32).T.reshape(1, _ROWS)
  seg1 = seg_ids.astype(jnp.int32).T.reshape(1, _ROWS)
  word2 = _pack_table(word_embed.T)
  pos3 = pos_embed.reshape(1, _MAXLEN * _EMBED)
  segtab2 = seg_embed.reshape(1, 2 * _EMBED)
  out2 = _embed_sc(ids1, seg1, word2, pos3, segtab2)
  return jnp.transpose(out2.reshape(_SEQ, _BATCH, _EMBED), (1, 0, 2))


# speedup vs baseline: 1.5585x; 1.0040x over previous
import functools

import jax
import jax.numpy as jnp
from jax import lax
from jax.experimental import pallas as pl
from jax.experimental.pallas import tpu as pltpu
from jax.experimental.pallas import tpu_sc as plsc

_VOCAB = 1000000
_EMBED = 64
_MAXLEN = 64
_SEGN = 2
_BATCH = 4096
_SEQ = 64

_WHALF = 512000
_BSTART = 488000
_TPB = 4096

_NC = 2
_NS = 16
_NW = _NC * _NS
_ROWS = _BATCH * _SEQ
_RPW = _ROWS // _NW
_C = 256
_NCHUNK = _RPW // _C
_G = 128
_NG = _C // _G


def _tp_body(a_ref, b_ref, o_ref):
  o_ref[:, 0:_EMBED] = a_ref[...].T
  o_ref[:, _EMBED:2 * _EMBED] = b_ref[...].T


def _pack_table(wt):
  nblk = _WHALF // _TPB
  wtb = lax.slice(wt, (0, _BSTART), (_EMBED, _VOCAB))
  return pl.pallas_call(
      _tp_body,
      grid=(nblk,),
      in_specs=[
          pl.BlockSpec((_EMBED, _TPB), lambda i: (0, i)),
          pl.BlockSpec((_EMBED, _TPB), lambda i: (0, i)),
      ],
      out_specs=pl.BlockSpec((_TPB, 2 * _EMBED), lambda i: (i, 0)),
      out_shape=jax.ShapeDtypeStruct((_WHALF, 2 * _EMBED), jnp.float32),
  )(wt, wtb)


def _sc_body(ids_hbm, seg_hbm, word_hbm, pos_hbm, segtab_hbm, out_hbm,
             idsw_v, segw_v, idxg0_v, idxg1_v, gbuf0_v, gbuf1_v, obuf_v,
             pos_v, segtab_v, sem0, sem1):
  idxg = (idxg0_v, idxg1_v)
  gbuf = (gbuf0_v, gbuf1_v)
  wid = lax.axis_index("c") * _NS + lax.axis_index("s")
  base = wid * _RPW

  t0 = pl.multiple_of(wid * _RPW, _RPW)
  pltpu.sync_copy(ids_hbm.at[:, pl.ds(t0, _RPW)], idsw_v)
  pltpu.sync_copy(seg_hbm.at[:, pl.ds(t0, _RPW)], segw_v)
  pltpu.sync_copy(pos_hbm.at[:, pl.ds(pl.multiple_of(wid * 2 * _EMBED, 128),
                                      2 * _EMBED)], pos_v)
  pltpu.sync_copy(segtab_hbm, segtab_v)

  seg0 = [segtab_v[0, pl.ds(o * 16, 16)] for o in range(4)]
  dseg = [segtab_v[0, pl.ds(_EMBED + o * 16, 16)] - seg0[o] for o in range(4)]

  def fill_idx(c, slot):
    for jr in range(_NG):
      def body(t, _, jr=jr):
        d = pl.ds(c * _C + jr * _G + t * 16, 16)
        v = idsw_v[0, d]
        idxg[slot][jr, pl.ds(t * 16, 16)] = jnp.where(
            v >= _WHALF, v - _BSTART, v)
        return _
      lax.fori_loop(0, _G // 16, body, None)

  def fire(c, slot, sem):
    fill_idx(c, slot)
    return [pltpu.async_copy(word_hbm.at[idxg[slot].at[j]],
                             gbuf[slot].at[pl.ds(j * _G, _G)], sem)
            for j in range(_NG)]

  def compute(c, slot):
    hp = lax.mul(lax.div(c, _NCHUNK // 2), _EMBED)
    basev = [pos_v[0, pl.ds(hp + o * 16, 16)] + seg0[o] for o in range(4)]

    def grp(t, _):
      d = pl.ds(c * _C + t * 16, 16)
      ids16 = idsw_v[0, d]
      seg16 = segw_v[0, d]
      for i in range(16):
        r = t * 16 + i
        h64 = lax.mul(lax.convert_element_type(ids16[i] >= _WHALF,
                                               jnp.int32), _EMBED)
        gf = lax.convert_element_type(seg16[i], jnp.float32)
        gfv = lax.broadcast(gf, (16,))
        q = t * 8 + (i // 2)
        ocol = (i % 2) * _EMBED
        for jj in range(4):
          o = jj * 16
          v = gbuf[slot][r, pl.ds(h64 + o, 16)]
          obuf_v[q, pl.ds(ocol + o, 16)] = v + basev[jj] + gfv * dseg[jj]
      return _
    lax.fori_loop(0, _C // 16, grp, None)

    ob = pl.multiple_of((base + c * _C) // 2, _C // 2)
    pltpu.sync_copy(obuf_v, out_hbm.at[pl.ds(ob, _C // 2)])

  def pair(p, _):
    c0 = p * 2
    cps0 = fire(c0, 0, sem0)
    cps1 = fire(c0 + 1, 1, sem1)
    for cp in cps0:
      cp.wait()
    compute(c0, 0)
    for cp in cps1:
      cp.wait()
    compute(c0 + 1, 1)
    return _

  lax.fori_loop(0, _NCHUNK // 2, pair, None)


@functools.partial(
    pl.kernel,
    out_type=jax.ShapeDtypeStruct((_ROWS // 2, 2 * _EMBED), jnp.float32),
    mesh=plsc.VectorSubcoreMesh(core_axis_name="c", subcore_axis_name="s"),
    scratch_types=[
        pltpu.VMEM((1, _RPW), jnp.int32),
        pltpu.VMEM((1, _RPW), jnp.int32),
        pltpu.VMEM((_NG, _G), jnp.int32),
        pltpu.VMEM((_NG, _G), jnp.int32),
        pltpu.VMEM((_C, 2 * _EMBED), jnp.float32),
        pltpu.VMEM((_C, 2 * _EMBED), jnp.float32),
        pltpu.VMEM((_C // 2, 2 * _EMBED), jnp.float32),
        pltpu.VMEM((1, 2 * _EMBED), jnp.float32),
        pltpu.VMEM((1, 2 * _EMBED), jnp.float32),
        pltpu.SemaphoreType.DMA,
        pltpu.SemaphoreType.DMA,
    ],
    compiler_params=pltpu.CompilerParams(use_tc_tiling_on_sc=True),
)
def _embed_sc(*refs):
  _sc_body(*refs)


@jax.jit
def kernel(input_ids, seg_ids, word_embed, pos_embed, seg_embed):
  ids1 = input_ids.astype(jnp.int32).T.reshape(1, _ROWS)
  seg1 = seg_ids.astype(jnp.int32).T.reshape(1, _ROWS)
  word2 = _pack_table(word_embed.T)
  pos3 = pos_embed.reshape(1, _MAXLEN * _EMBED)
  segtab2 = seg_embed.reshape(1, 2 * _EMBED)
  out2 = _embed_sc(ids1, seg1, word2, pos3, segtab2)
  return jnp.transpose(out2.reshape(_SEQ, _BATCH, _EMBED), (1, 0, 2))
